# baseline (speedup 1.0000x reference)
import jax
import jax.numpy as jnp
from jax.experimental import pallas as pl
from jax.experimental.pallas import tpu as pltpu

_N, _C, _P = 1024, 256, 784
_BN = 8
_GRID = _N // _BN


def _body(z_ref, snr_ref, w1_ref, b1_ref, g1_ref, be1_ref, w2_ref, b2_ref,
          g2_ref, be2_ref, w3_ref, b3_ref, gmb_ref, tmp_ref,
          hard_ref, soft_ref, logits_ref, feat_ref):
    i = pl.program_id(0)
    blk = z_ref[...]
    s = jnp.sum(blk, axis=-1) / float(_P)
    feat_ref[pl.ds(i * _BN, _BN), :] = s

    @pl.when(i == _GRID - 1)
    def _tail():
        feat = feat_ref[...]
        w1 = w1_ref[...]
        h = jax.lax.dot_general(feat, w1[:, :_C], (((1,), (1,)), ((), ())),
                                preferred_element_type=jnp.float32)
        h = h + snr_ref[...] * w1[:, _C][None, :] + b1_ref[...]
        h = jnp.maximum(h, 0.0)
        mu = jnp.mean(h, axis=0, keepdims=True)
        var = jnp.mean((h - mu) ** 2, axis=0, keepdims=True)
        h = g1_ref[...] * (h - mu) / jnp.sqrt(var + 1e-5) + be1_ref[...]
        h = jax.lax.dot_general(h, w2_ref[...], (((1,), (1,)), ((), ())),
                                preferred_element_type=jnp.float32)
        h = h + b2_ref[...]
        h = jnp.maximum(h, 0.0)
        mu2 = jnp.mean(h, axis=0, keepdims=True)
        var2 = jnp.mean((h - mu2) ** 2, axis=0, keepdims=True)
        h = g2_ref[...] * (h - mu2) / jnp.sqrt(var2 + 1e-5) + be2_ref[...]
        logits = jax.lax.dot_general(h, w3_ref[...], (((1,), (1,)), ((), ())),
                                     preferred_element_type=jnp.float32)
        logits = logits + b3_ref[...]
        logits_ref[...] = logits

        y = (logits + gmb_ref[...]) / tmp_ref[...]
        m = jnp.max(y, axis=-1, keepdims=True)
        e = jnp.exp(y - m)
        soft = e / jnp.sum(e, axis=-1, keepdims=True)

        best = soft[:, 0:1]
        idx = jnp.zeros((_N, 1), jnp.int32)
        for k in range(1, 7):
            ck = soft[:, k:k + 1]
            p = ck > best
            best = jnp.where(p, ck, best)
            idx = jnp.where(p, k, idx)
        jj = jax.lax.broadcasted_iota(jnp.int32, (_N, 6), 1) + 1
        hard_ref[...] = (idx >= jj).astype(jnp.float32)

        tri_r = jax.lax.broadcasted_iota(jnp.int32, (6, 6), 0)
        tri_c = jax.lax.broadcasted_iota(jnp.int32, (6, 6), 1)
        tri = (tri_r >= tri_c).astype(jnp.float32)
        soft_ref[...] = jax.lax.dot_general(soft[:, 1:], tri,
                                            (((1,), (0,)), ((), ())),
                                            preferred_element_type=jnp.float32)


def _full(shape):
    return pl.BlockSpec(shape, lambda i: (0,) * len(shape))


def kernel(z, SNR, W1, b1, g1, be1, W2, b2, g2, be2, W3, b3, temp):
    zr = z.reshape(_N, _C, _P)
    gmb = jax.random.gumbel(jax.random.key(42), (_N, 7), dtype=jnp.float32)
    tmp = jnp.reshape(jnp.asarray(temp, jnp.float32), (1, 1))
    hard, soft, logits = pl.pallas_call(
        _body,
        grid=(_GRID,),
        in_specs=[pl.BlockSpec((_BN, _C, _P), lambda i: (i, 0, 0)),
                  _full((_N, 1)), _full((64, _C + 1)),
                  _full((1, 64)), _full((1, 64)), _full((1, 64)),
                  _full((64, 64)), _full((1, 64)), _full((1, 64)),
                  _full((1, 64)), _full((7, 64)), _full((1, 7)),
                  _full((_N, 7)), _full((1, 1))],
        out_specs=(_full((_N, 6)), _full((_N, 6)), _full((_N, 7))),
        out_shape=(jax.ShapeDtypeStruct((_N, 6), jnp.float32),
                   jax.ShapeDtypeStruct((_N, 6), jnp.float32),
                   jax.ShapeDtypeStruct((_N, 7), jnp.float32)),
        scratch_shapes=[pltpu.VMEM((_N, _C), jnp.float32)],
        compiler_params=pltpu.CompilerParams(
            dimension_semantics=("arbitrary",)),
    )(zr, SNR, W1, b1.reshape(1, 64), g1.reshape(1, 64), be1.reshape(1, 64),
      W2, b2.reshape(1, 64), g2.reshape(1, 64), be2.reshape(1, 64),
      W3, b3.reshape(1, 7), gmb, tmp)
    return (hard, soft, logits)


# speedup vs baseline: 4.1042x; 4.1042x over previous
import jax
import jax.numpy as jnp
from jax.experimental import pallas as pl
from jax.experimental.pallas import tpu as pltpu

_N, _C, _H, _W = 1024, 256, 28, 28
_P = _H * _W
_BN = 8
_GRID = _N // _BN


def _mean_body(zt_ref, out_ref):
    acc = None
    for wi in range(7):
        for wj in range(7):
            w = None
            for j in range(4):
                for i in range(4):
                    p = (4 * wi + i) * _W + (4 * wj + j)
                    t = zt_ref[p]
                    w = t if w is None else w + t
            acc = w if acc is None else acc + w
    out_ref[...] = acc * jnp.float32(1.0 / _P)


def _spatial_mean(z):
    zt = jnp.transpose(z, (2, 3, 0, 1)).reshape(_P, _N, _C)
    return pl.pallas_call(
        _mean_body,
        grid=(_GRID,),
        in_specs=[pl.BlockSpec((_P, _BN, _C), lambda i: (0, i, 0))],
        out_specs=pl.BlockSpec((_BN, _C), lambda i: (i, 0)),
        out_shape=jax.ShapeDtypeStruct((_N, _C), jnp.float32),
        compiler_params=pltpu.CompilerParams(
            dimension_semantics=("arbitrary",)),
    )(zt)


def _thermo(h):
    h = jnp.flip(h, -1)
    s = jnp.cumsum(h, -1)
    return jnp.flip(s, -1)


def _bn_train(x, gamma, beta, eps=1e-5):
    mu = x.mean(0)
    var = x.var(0)
    return gamma * (x - mu) / jnp.sqrt(var + eps) + beta


def kernel(z, SNR, W1, b1, g1, be1, W2, b2, g2, be2, W3, b3, temp):
    feat = jnp.concatenate([_spatial_mean(z), SNR], axis=-1)
    h = feat @ W1.T + b1
    h = jax.nn.relu(h)
    h = _bn_train(h, g1, be1)
    h = h @ W2.T + b2
    h = jax.nn.relu(h)
    h = _bn_train(h, g2, be2)
    logits = h @ W3.T + b3
    g = jax.random.gumbel(jax.random.key(42), logits.shape, dtype=logits.dtype)
    soft = jax.nn.softmax((logits + g) / temp, axis=-1)
    index = jax.nn.one_hot(jnp.argmax(soft, axis=-1), soft.shape[-1], dtype=soft.dtype)
    bias = jax.lax.stop_gradient(index - soft)
    hard = soft + bias
    soft_mask = _thermo(soft[:, 1:])
    hard_mask = _thermo(hard[:, 1:])
    return (hard_mask, soft_mask, logits)


# speedup vs baseline: 4.1531x; 1.0119x over previous
import jax
import jax.numpy as jnp
from jax.experimental import pallas as pl
from jax.experimental.pallas import tpu as pltpu

_N, _C, _H, _W = 1024, 256, 28, 28
_P = _H * _W
_BN = 16
_GRID = _N // _BN


def _mean_body(zt_ref, out_ref):
    acc = None
    for wi in range(7):
        for wj in range(7):
            w = None
            for j in range(4):
                for i in range(4):
                    p = (4 * wi + i) * _W + (4 * wj + j)
                    t = zt_ref[p]
                    w = t if w is None else w + t
            acc = w if acc is None else acc + w
    out_ref[...] = acc * jnp.float32(1.0 / _P)


def _spatial_mean(z):
    zt = jnp.transpose(z, (2, 3, 0, 1)).reshape(_P, _N, _C)
    return pl.pallas_call(
        _mean_body,
        grid=(_GRID,),
        in_specs=[pl.BlockSpec((_P, _BN, _C), lambda i: (0, i, 0))],
        out_specs=pl.BlockSpec((_BN, _C), lambda i: (i, 0)),
        out_shape=jax.ShapeDtypeStruct((_N, _C), jnp.float32),
        compiler_params=pltpu.CompilerParams(
            dimension_semantics=("arbitrary",)),
    )(zt)


def _thermo(h):
    h = jnp.flip(h, -1)
    s = jnp.cumsum(h, -1)
    return jnp.flip(s, -1)


def _bn_train(x, gamma, beta, eps=1e-5):
    mu = x.mean(0)
    var = x.var(0)
    return gamma * (x - mu) / jnp.sqrt(var + eps) + beta


def kernel(z, SNR, W1, b1, g1, be1, W2, b2, g2, be2, W3, b3, temp):
    feat = jnp.concatenate([_spatial_mean(z), SNR], axis=-1)
    h = feat @ W1.T + b1
    h = jax.nn.relu(h)
    h = _bn_train(h, g1, be1)
    h = h @ W2.T + b2
    h = jax.nn.relu(h)
    h = _bn_train(h, g2, be2)
    logits = h @ W3.T + b3
    g = jax.random.gumbel(jax.random.key(42), logits.shape, dtype=logits.dtype)
    soft = jax.nn.softmax((logits + g) / temp, axis=-1)
    index = jax.nn.one_hot(jnp.argmax(soft, axis=-1), soft.shape[-1], dtype=soft.dtype)
    bias = jax.lax.stop_gradient(index - soft)
    hard = soft + bias
    soft_mask = _thermo(soft[:, 1:])
    hard_mask = _thermo(hard[:, 1:])
    return (hard_mask, soft_mask, logits)


# speedup vs baseline: 4.2244x; 1.0172x over previous
import jax
import jax.numpy as jnp
from jax.experimental import pallas as pl
from jax.experimental.pallas import tpu as pltpu

_N, _C, _H, _W = 1024, 256, 28, 28
_P = _H * _W
_BN = 16
_GRID = _N // _BN


def _mean_body(zt_ref, out_ref):
    w = None
    for j in range(4):
        for i in range(4):
            t = zt_ref[i, j]
            w = t if w is None else w + t
    wi, wj = pl.program_id(0), pl.program_id(1)
    first = (wi == 0) & (wj == 0)
    last = (wi == 6) & (wj == 6)

    @pl.when(first)
    def _init():
        out_ref[...] = w

    @pl.when(~first & ~last)
    def _accum():
        out_ref[...] = out_ref[...] + w

    @pl.when(last)
    def _final():
        out_ref[...] = (out_ref[...] + w) * jnp.float32(1.0 / _P)


def _spatial_mean(z):
    zt = jnp.transpose(z, (2, 3, 0, 1))
    return pl.pallas_call(
        _mean_body,
        grid=(7, 7),
        in_specs=[pl.BlockSpec((4, 4, _N, _C), lambda a, b: (a, b, 0, 0))],
        out_specs=pl.BlockSpec((_N, _C), lambda a, b: (0, 0)),
        out_shape=jax.ShapeDtypeStruct((_N, _C), jnp.float32),
        compiler_params=pltpu.CompilerParams(
            dimension_semantics=("arbitrary", "arbitrary")),
    )(zt)


def _thermo(h):
    h = jnp.flip(h, -1)
    s = jnp.cumsum(h, -1)
    return jnp.flip(s, -1)


def _bn_train(x, gamma, beta, eps=1e-5):
    mu = x.mean(0)
    var = x.var(0)
    return gamma * (x - mu) / jnp.sqrt(var + eps) + beta


def kernel(z, SNR, W1, b1, g1, be1, W2, b2, g2, be2, W3, b3, temp):
    feat = jnp.concatenate([_spatial_mean(z), SNR], axis=-1)
    h = feat @ W1.T + b1
    h = jax.nn.relu(h)
    h = _bn_train(h, g1, be1)
    h = h @ W2.T + b2
    h = jax.nn.relu(h)
    h = _bn_train(h, g2, be2)
    logits = h @ W3.T + b3
    g = jax.random.gumbel(jax.random.key(42), logits.shape, dtype=logits.dtype)
    soft = jax.nn.softmax((logits + g) / temp, axis=-1)
    index = jax.nn.one_hot(jnp.argmax(soft, axis=-1), soft.shape[-1], dtype=soft.dtype)
    bias = jax.lax.stop_gradient(index - soft)
    hard = soft + bias
    soft_mask = _thermo(soft[:, 1:])
    hard_mask = _thermo(hard[:, 1:])
    return (hard_mask, soft_mask, logits)
